# baseline (speedup 1.0000x reference)
import functools

import jax
import jax.numpy as jnp
from jax import lax
from jax.experimental import pallas as pl
from jax.experimental.pallas import tpu as pltpu
from jax.experimental.pallas import tpu_sc as plsc

N = 10000
D = 128
H = 128
NC = 2
NS = 16
NW = NC * NS
CHUNK = 128
SEG = 8
NSEG = 320
E_PAD = NSEG * SEG * CHUNK
SEG_A = 17
SEG_B = 3
ACC_ROWS = 10112
ZROWS = ACC_ROWS // NS
NB = 2
LOOK = 1

RB = 1000
GRID = N // RB


def _agg_body(h_hbm, src_hbm, dst_hbm, out_hbm, src_v, dst_v,
              rows0, rows1, acc_s, sg0, sg1, ss0, ss1):
    rows = (rows0, rows1)
    semg = (sg0, sg1)
    sems = (ss0, ss1)
    c = lax.axis_index("c")
    s = lax.axis_index("s")

    def zrow(i, carry):
        for q in range(D // 16):
            rows0[i, pl.ds(q * 16, 16)] = jnp.zeros((16,), jnp.float32)
        return carry

    lax.fori_loop(0, CHUNK, zrow, 0)

    zbase = s * ZROWS

    def zcopy(r, carry):
        pltpu.sync_copy(rows0, acc_s.at[pl.ds(zbase + r * CHUNK, CHUNK)])
        return carry

    lax.fori_loop(0, ZROWS // CHUNK, zcopy, 0)
    pltpu.sync_copy(rows0.at[pl.ds(0, ZROWS % CHUNK)],
                    acc_s.at[pl.ds(zbase + (ZROWS // CHUNK) * CHUNK,
                                   ZROWS % CHUNK)])

    plsc.subcore_barrier()

    def run_segment(seg, carry):
        def start_g(jj, b):
            pltpu.async_copy(h_hbm.at[src_v.at[jj]], rows[b], semg[b % 2])

        def wait_g(jj, b):
            pltpu.make_async_copy(h_hbm.at[src_v.at[jj]], rows[b],
                                  semg[b % 2]).wait()

        def start_s(jj, b):
            pltpu.async_copy(rows[b], acc_s.at[dst_v.at[jj]],
                             sems[b % 2], add=True)

        def wait_s(jj, b):
            pltpu.make_async_copy(rows[b], acc_s.at[dst_v.at[jj]],
                                  sems[b % 2]).wait()

        pltpu.sync_copy(src_hbm.at[seg], src_v)
        pltpu.sync_copy(dst_hbm.at[seg], dst_v)
        for b in range(LOOK):
            start_g(b, b)

        def group(g, carry2):
            for b in range(NB):
                jj = g * NB + b
                wait_g(jj, b)

                @pl.when(jj >= LOOK)
                def _():
                    wait_s(jj - LOOK, (b + NB - LOOK) % NB)

                start_s(jj, b)

                @pl.when(jj < SEG - LOOK)
                def _():
                    start_g(jj + LOOK, (b + LOOK) % NB)
            return carry2

        lax.fori_loop(0, SEG // NB, group, 0)
        for jj in range(SEG - LOOK, SEG):
            wait_s(jj, jj % NB)
        return carry

    nseg = jnp.where(c == 0, SEG_A, SEG_B)
    seg0 = jnp.where(c == 0, s * SEG_A, NS * SEG_A + s * SEG_B)
    lax.fori_loop(0, nseg, lambda k, cr: run_segment(seg0 + k, cr), 0)

    plsc.subcore_barrier()

    pltpu.sync_copy(acc_s.at[pl.ds(s * ZROWS, ZROWS)],
                    out_hbm.at[c, pl.ds(s * ZROWS, ZROWS)])


def _make_agg():
    mesh = plsc.VectorSubcoreMesh(core_axis_name="c", subcore_axis_name="s",
                                  num_cores=NC, num_subcores=NS)
    return functools.partial(
        pl.kernel,
        out_type=jax.ShapeDtypeStruct((NC, ACC_ROWS, D), jnp.float32),
        mesh=mesh,
        scratch_types=[
            pltpu.VMEM((SEG, CHUNK), jnp.int32),
            pltpu.VMEM((SEG, CHUNK), jnp.int32),
            pltpu.VMEM((CHUNK, D), jnp.float32),
            pltpu.VMEM((CHUNK, D), jnp.float32),
            pltpu.VMEM_SHARED((ACC_ROWS, D), jnp.float32),
            pltpu.SemaphoreType.DMA,
            pltpu.SemaphoreType.DMA,
            pltpu.SemaphoreType.DMA,
            pltpu.SemaphoreType.DMA,
        ],
    )(_agg_body)


def _mlp_body(h_ref, a0_ref, a1_ref, w1_ref, b1_ref, w2_ref, b2_ref, o_ref):
    z = h_ref[...] + a0_ref[0] + a1_ref[0]
    z = lax.dot_general(z, w1_ref[...], (((1,), (0,)), ((), ())),
                        precision=lax.Precision.HIGHEST,
                        preferred_element_type=jnp.float32)
    z = jnp.maximum(z + b1_ref[...], 0.0)
    z = lax.dot_general(z, w2_ref[...], (((1,), (0,)), ((), ())),
                        precision=lax.Precision.HIGHEST,
                        preferred_element_type=jnp.float32)
    o_ref[...] = jnp.maximum(z + b2_ref[...], 0.0)


def _mlp(h, agg, w1, b1, w2, b2):
    row_spec = pl.BlockSpec((RB, D), lambda i: (i, 0))
    a0_spec = pl.BlockSpec((1, RB, D), lambda i: (0, i, 0))
    a1_spec = pl.BlockSpec((1, RB, D), lambda i: (1, i, 0))
    full = pl.BlockSpec((D, H), lambda i: (0, 0))
    bias = pl.BlockSpec((1, H), lambda i: (0, 0))
    return pl.pallas_call(
        _mlp_body,
        grid=(GRID,),
        in_specs=[row_spec, a0_spec, a1_spec, full, bias, full, bias],
        out_specs=pl.BlockSpec((RB, H), lambda i: (i, 0)),
        out_shape=jax.ShapeDtypeStruct((N, H), jnp.float32),
    )(h, agg, agg, w1, b1.reshape(1, H), w2, b2.reshape(1, H))


def kernel(x, edge_index, W1a, b1a, W2a, b2a, W1b, b1b, W2b, b2b):
    src = edge_index[0]
    dst = edge_index[1]
    pad = E_PAD - src.shape[0]
    src_p = jnp.concatenate(
        [src, jnp.zeros((pad,), jnp.int32)]).reshape(NSEG, SEG, CHUNK)
    dst_p = jnp.concatenate(
        [dst, jnp.full((pad,), N, jnp.int32)]).reshape(NSEG, SEG, CHUNK)

    agg_fn = _make_agg()
    agg = agg_fn(x, src_p, dst_p)
    h1 = _mlp(x, agg, W1a, b1a, W2a, b2a)
    agg2 = agg_fn(h1, src_p, dst_p)
    h2 = _mlp(h1, agg2, W1b, b1b, W2b, b2b)
    return h2


# speedup vs baseline: 1.1985x; 1.1985x over previous
import functools

import jax
import jax.numpy as jnp
from jax import lax
from jax.experimental import pallas as pl
from jax.experimental.pallas import tpu as pltpu
from jax.experimental.pallas import tpu_sc as plsc

N = 10000
D = 128
H = 128
NC = 2
NS = 16
NW = NC * NS
CHUNK = 128
SEG = 16
NSEG = 160
E_PAD = NSEG * SEG * CHUNK
SEG_A = 9
SEG_B = 1
ACC_ROWS = 10112
ZROWS = ACC_ROWS // NS
NB = 2
LOOK = 1

RB = 1000
GRID = N // RB


def _agg_body(h_hbm, src_hbm, dst_hbm, out_hbm, src_v, dst_v,
              rows0, rows1, acc_s, sg0, sg1, ss0, ss1):
    rows = (rows0, rows1)
    semg = (sg0, sg1)
    sems = (ss0, ss1)
    c = lax.axis_index("c")
    s = lax.axis_index("s")

    def zrow(i, carry):
        for q in range(D // 16):
            rows0[i, pl.ds(q * 16, 16)] = jnp.zeros((16,), jnp.float32)
        return carry

    lax.fori_loop(0, CHUNK, zrow, 0)

    zbase = s * ZROWS

    def zcopy(r, carry):
        pltpu.sync_copy(rows0, acc_s.at[pl.ds(zbase + r * CHUNK, CHUNK)])
        return carry

    lax.fori_loop(0, ZROWS // CHUNK, zcopy, 0)
    pltpu.sync_copy(rows0.at[pl.ds(0, ZROWS % CHUNK)],
                    acc_s.at[pl.ds(zbase + (ZROWS // CHUNK) * CHUNK,
                                   ZROWS % CHUNK)])

    plsc.subcore_barrier()

    def run_segment(seg, carry):
        def start_g(jj, b):
            pltpu.async_copy(h_hbm.at[src_v.at[jj]], rows[b], semg[b % 2])

        def wait_g(jj, b):
            pltpu.make_async_copy(h_hbm.at[src_v.at[jj]], rows[b],
                                  semg[b % 2]).wait()

        def start_s(jj, b):
            pltpu.async_copy(rows[b], acc_s.at[dst_v.at[jj]],
                             sems[b % 2], add=True)

        def wait_s(jj, b):
            pltpu.make_async_copy(rows[b], acc_s.at[dst_v.at[jj]],
                                  sems[b % 2]).wait()

        pltpu.sync_copy(src_hbm.at[seg], src_v)
        pltpu.sync_copy(dst_hbm.at[seg], dst_v)
        for b in range(LOOK):
            start_g(b, b)

        def group(g, carry2):
            for b in range(NB):
                jj = g * NB + b
                wait_g(jj, b)

                @pl.when(jj >= LOOK)
                def _():
                    wait_s(jj - LOOK, (b + NB - LOOK) % NB)

                start_s(jj, b)

                @pl.when(jj < SEG - LOOK)
                def _():
                    start_g(jj + LOOK, (b + LOOK) % NB)
            return carry2

        lax.fori_loop(0, SEG // NB, group, 0)
        for jj in range(SEG - LOOK, SEG):
            wait_s(jj, jj % NB)
        return carry

    nseg = jnp.where(c == 0, SEG_A, SEG_B)
    seg0 = jnp.where(c == 0, s * SEG_A, NS * SEG_A + s * SEG_B)
    lax.fori_loop(0, nseg, lambda k, cr: run_segment(seg0 + k, cr), 0)

    plsc.subcore_barrier()

    pltpu.sync_copy(acc_s.at[pl.ds(s * ZROWS, ZROWS)],
                    out_hbm.at[c, pl.ds(s * ZROWS, ZROWS)])


def _make_agg():
    mesh = plsc.VectorSubcoreMesh(core_axis_name="c", subcore_axis_name="s",
                                  num_cores=NC, num_subcores=NS)
    return functools.partial(
        pl.kernel,
        out_type=jax.ShapeDtypeStruct((NC, ACC_ROWS, D), jnp.float32),
        mesh=mesh,
        scratch_types=[
            pltpu.VMEM((SEG, CHUNK), jnp.int32),
            pltpu.VMEM((SEG, CHUNK), jnp.int32),
            pltpu.VMEM((CHUNK, D), jnp.float32),
            pltpu.VMEM((CHUNK, D), jnp.float32),
            pltpu.VMEM_SHARED((ACC_ROWS, D), jnp.float32),
            pltpu.SemaphoreType.DMA,
            pltpu.SemaphoreType.DMA,
            pltpu.SemaphoreType.DMA,
            pltpu.SemaphoreType.DMA,
        ],
    )(_agg_body)


def _mlp_body(h_ref, a0_ref, a1_ref, w1_ref, b1_ref, w2_ref, b2_ref, o_ref):
    z = h_ref[...] + a0_ref[0] + a1_ref[0]
    z = lax.dot_general(z, w1_ref[...], (((1,), (0,)), ((), ())),
                        preferred_element_type=jnp.float32)
    z = jnp.maximum(z + b1_ref[...], 0.0)
    z = lax.dot_general(z, w2_ref[...], (((1,), (0,)), ((), ())),
                        preferred_element_type=jnp.float32)
    o_ref[...] = jnp.maximum(z + b2_ref[...], 0.0)


def _mlp(h, agg, w1, b1, w2, b2):
    row_spec = pl.BlockSpec((RB, D), lambda i: (i, 0))
    a0_spec = pl.BlockSpec((1, RB, D), lambda i: (0, i, 0))
    a1_spec = pl.BlockSpec((1, RB, D), lambda i: (1, i, 0))
    full = pl.BlockSpec((D, H), lambda i: (0, 0))
    bias = pl.BlockSpec((1, H), lambda i: (0, 0))
    return pl.pallas_call(
        _mlp_body,
        grid=(GRID,),
        in_specs=[row_spec, a0_spec, a1_spec, full, bias, full, bias],
        out_specs=pl.BlockSpec((RB, H), lambda i: (i, 0)),
        out_shape=jax.ShapeDtypeStruct((N, H), jnp.float32),
    )(h, agg, agg, w1, b1.reshape(1, H), w2, b2.reshape(1, H))


def kernel(x, edge_index, W1a, b1a, W2a, b2a, W1b, b1b, W2b, b2b):
    src = edge_index[0]
    dst = edge_index[1]
    pad = E_PAD - src.shape[0]
    src_p = jnp.concatenate(
        [src, jnp.zeros((pad,), jnp.int32)]).reshape(NSEG, SEG, CHUNK)
    dst_p = jnp.concatenate(
        [dst, jnp.full((pad,), N, jnp.int32)]).reshape(NSEG, SEG, CHUNK)

    agg_fn = _make_agg()
    agg = agg_fn(x, src_p, dst_p)
    h1 = _mlp(x, agg, W1a, b1a, W2a, b2a)
    agg2 = agg_fn(h1, src_p, dst_p)
    h2 = _mlp(h1, agg2, W1b, b1b, W2b, b2b)
    return h2
